# Initial kernel scaffold; baseline (speedup 1.0000x reference)
import jax
import jax.numpy as jnp
from jax.experimental import pallas as pl


def kernel(node_feat, edge_index, edge_feat, W_edge, b_edge, W_ih, W_hh, b_ih, b_hh):
    raise NotImplementedError("write your pallas kernel here")


# speedup vs baseline: 9.2974x; 9.2974x over previous
import functools

import jax
import jax.numpy as jnp
from jax import lax
from jax.experimental import pallas as pl
from jax.experimental.pallas import tpu as pltpu
from jax.experimental.pallas import tpu_sc as plsc

_N = 10000
_E = 320000
_D = 128
_NT = 4
_STEPS = 4

_NTILES = 32
_EPT = _E // _NTILES
_C = 80
_NCH = _EPT // _C
_NP = 10240
_RPT = _NP // 16
_NSEG = 5
_CPS = _NCH // _NSEG


def _gidx_body(ef_ref, src_ref, o_ref):
    best = ef_ref[0]
    idx = jnp.zeros_like(src_ref[...])
    for i in range(1, _NT):
        c = ef_ref[i]
        m = c > best
        idx = jnp.where(m, i, idx)
        best = jnp.where(m, c, best)
    o_ref[...] = idx * _N + src_ref[...]


def _gidx_call(ef_t, src2, interpret=False):
    return pl.pallas_call(
        _gidx_body,
        out_shape=jax.ShapeDtypeStruct((_E // _D, _D), jnp.int32),
        interpret=interpret,
    )(ef_t, src2)


def _y_body(h_ref, w_ref, b_ref, y_ref):
    y_ref[0] = (
        jnp.dot(h_ref[...], w_ref[0], preferred_element_type=jnp.float32)
        + b_ref[0]
    )


def _y_call(h, wt, b3, interpret=False):
    return pl.pallas_call(
        _y_body,
        grid=(_NT,),
        in_specs=[
            pl.BlockSpec((_N, _D), lambda i: (0, 0)),
            pl.BlockSpec((1, _D, _D), lambda i: (i, 0, 0)),
            pl.BlockSpec((1, 1, _D), lambda i: (i, 0, 0)),
        ],
        out_specs=pl.BlockSpec((1, _N, _D), lambda i: (i, 0, 0)),
        out_shape=jax.ShapeDtypeStruct((_NT, _N, _D), jnp.float32),
        interpret=interpret,
    )(h, wt, b3)


_BN = 2000


def _gru_body(p_ref, h_ref, wi_ref, wh_ref, bi_ref, bh_ref, o_ref):
    agg = p_ref[0] + p_ref[1]
    h = h_ref[...]
    gi = jnp.dot(agg, wi_ref[...], preferred_element_type=jnp.float32) + bi_ref[...]
    gh = jnp.dot(h, wh_ref[...], preferred_element_type=jnp.float32) + bh_ref[...]
    r = jax.nn.sigmoid(gi[:, :_D] + gh[:, :_D])
    z = jax.nn.sigmoid(gi[:, _D:2 * _D] + gh[:, _D:2 * _D])
    n = jnp.tanh(gi[:, 2 * _D:] + r * gh[:, 2 * _D:])
    o_ref[...] = (1.0 - z) * n + z * h


def _gru_call(parts, h, wiT, whT, bi2, bh2, interpret=False):
    return pl.pallas_call(
        _gru_body,
        grid=(_N // _BN,),
        in_specs=[
            pl.BlockSpec((2, _BN, _D), lambda i: (0, i, 0)),
            pl.BlockSpec((_BN, _D), lambda i: (i, 0)),
            pl.BlockSpec((_D, 3 * _D), lambda i: (0, 0)),
            pl.BlockSpec((_D, 3 * _D), lambda i: (0, 0)),
            pl.BlockSpec((1, 3 * _D), lambda i: (0, 0)),
            pl.BlockSpec((1, 3 * _D), lambda i: (0, 0)),
        ],
        out_specs=pl.BlockSpec((_BN, _D), lambda i: (i, 0)),
        out_shape=jax.ShapeDtypeStruct((_N, _D), jnp.float32),
        interpret=interpret,
    )(parts, h, wiT, whT, bi2, bh2)


@functools.lru_cache(maxsize=1)
def _make_sc_aggregate():
    mesh = plsc.VectorSubcoreMesh(core_axis_name="c", subcore_axis_name="s")

    @functools.partial(
        pl.kernel,
        out_type=jax.ShapeDtypeStruct((2, _NP, _D), jnp.float32),
        mesh=mesh,
        scratch_types=[
            pltpu.VMEM((_CPS, _C), jnp.int32),
            pltpu.VMEM((_CPS, _C), jnp.int32),
            pltpu.VMEM((_C, _D), jnp.float32),
            pltpu.VMEM_SHARED((_NP, _D), jnp.float32),
            pltpu.SemaphoreType.DMA,
        ],
    )
    def _sc_aggregate(y_hbm, gidx_hbm, dst_hbm, zero_hbm, out_hbm,
                      gidx_v, dst_v, rows_v, acc_sh, sem):
        cid = lax.axis_index("c")
        sid = lax.axis_index("s")
        tid = cid * 16 + sid

        pltpu.sync_copy(zero_hbm, rows_v)
        for z in range(_RPT // _C):
            pltpu.sync_copy(rows_v, acc_sh.at[pl.ds(sid * _RPT + z * _C, _C)])
        plsc.subcore_barrier()

        for seg in range(_NSEG):
            pltpu.sync_copy(gidx_hbm.at[tid, seg], gidx_v)
            pltpu.sync_copy(dst_hbm.at[tid, seg], dst_v)

            def body(j, carry):
                pltpu.async_copy(y_hbm.at[gidx_v.at[j]], rows_v, sem).wait()
                pltpu.sync_copy(rows_v, acc_sh.at[dst_v.at[j]], add=True)
                return carry

            lax.fori_loop(0, _CPS, body, 0)
        plsc.subcore_barrier()

        pltpu.sync_copy(acc_sh.at[pl.ds(sid * _RPT, _RPT)],
                        out_hbm.at[cid, pl.ds(sid * _RPT, _RPT)])

    return _sc_aggregate


def kernel(node_feat, edge_index, edge_feat, W_edge, b_edge,
           W_ih, W_hh, b_ih, b_hh):
    src = edge_index[0].astype(jnp.int32)
    dst = edge_index[1].astype(jnp.int32)

    ef_t = jnp.transpose(edge_feat).reshape(_NT, _E // _D, _D)
    src2 = src.reshape(_E // _D, _D)
    gidx4 = _gidx_call(ef_t, src2).reshape(_NTILES, _NSEG, _CPS, _C)
    dst4 = dst.reshape(_NTILES, _NSEG, _CPS, _C)

    wt = jnp.transpose(W_edge, (0, 2, 1))
    b3 = b_edge.reshape(_NT, 1, _D)
    wiT = jnp.transpose(W_ih)
    whT = jnp.transpose(W_hh)
    bi2 = b_ih.reshape(1, 3 * _D)
    bh2 = b_hh.reshape(1, 3 * _D)
    zeros = jnp.zeros((_C, _D), jnp.float32)

    h = node_feat
    for _ in range(_STEPS):
        y = _y_call(h, wt, b3).reshape(_NT * _N, _D)
        parts = _make_sc_aggregate()(y, gidx4, dst4, zeros)
        h = _gru_call(parts, h, wiT, whT, bi2, bh2)
    return h


# speedup vs baseline: 15.3214x; 1.6479x over previous
import functools

import jax
import jax.numpy as jnp
from jax import lax
from jax.experimental import pallas as pl
from jax.experimental.pallas import tpu as pltpu
from jax.experimental.pallas import tpu_sc as plsc

_N = 10000
_E = 320000
_D = 128
_NT = 4
_STEPS = 4

_NTILES = 32
_EPT = _E // _NTILES
_C = 80
_NCH = _EPT // _C
_NP = 10240
_RPT = _NP // 16
_NSEG = 5
_CPS = _NCH // _NSEG


def _gidx_body(ef_ref, src_ref, o_ref):
    best = ef_ref[0]
    idx = jnp.zeros_like(src_ref[...])
    for i in range(1, _NT):
        c = ef_ref[i]
        m = c > best
        idx = jnp.where(m, i, idx)
        best = jnp.where(m, c, best)
    o_ref[...] = idx * _N + src_ref[...]


def _gidx_call(ef_t, src2, interpret=False):
    return pl.pallas_call(
        _gidx_body,
        out_shape=jax.ShapeDtypeStruct((_E // _D, _D), jnp.int32),
        interpret=interpret,
    )(ef_t, src2)


def _y_body(h_ref, w_ref, b_ref, y_ref):
    y_ref[0] = (
        jnp.dot(h_ref[...], w_ref[0], preferred_element_type=jnp.float32)
        + b_ref[0]
    )


def _y_call(h, wt, b3, interpret=False):
    return pl.pallas_call(
        _y_body,
        grid=(_NT,),
        in_specs=[
            pl.BlockSpec((_N, _D), lambda i: (0, 0)),
            pl.BlockSpec((1, _D, _D), lambda i: (i, 0, 0)),
            pl.BlockSpec((1, 1, _D), lambda i: (i, 0, 0)),
        ],
        out_specs=pl.BlockSpec((1, _N, _D), lambda i: (i, 0, 0)),
        out_shape=jax.ShapeDtypeStruct((_NT, _N, _D), jnp.float32),
        interpret=interpret,
    )(h, wt, b3)


_BN = 2000


def _gru_body(p_ref, h_ref, wi_ref, wh_ref, bi_ref, bh_ref, o_ref):
    agg = p_ref[0] + p_ref[1]
    h = h_ref[...]
    gi = jnp.dot(agg, wi_ref[...], preferred_element_type=jnp.float32) + bi_ref[...]
    gh = jnp.dot(h, wh_ref[...], preferred_element_type=jnp.float32) + bh_ref[...]
    r = jax.nn.sigmoid(gi[:, :_D] + gh[:, :_D])
    z = jax.nn.sigmoid(gi[:, _D:2 * _D] + gh[:, _D:2 * _D])
    n = jnp.tanh(gi[:, 2 * _D:] + r * gh[:, 2 * _D:])
    o_ref[...] = (1.0 - z) * n + z * h


def _gru_call(parts, h, wiT, whT, bi2, bh2, interpret=False):
    return pl.pallas_call(
        _gru_body,
        grid=(_N // _BN,),
        in_specs=[
            pl.BlockSpec((2, _BN, _D), lambda i: (0, i, 0)),
            pl.BlockSpec((_BN, _D), lambda i: (i, 0)),
            pl.BlockSpec((_D, 3 * _D), lambda i: (0, 0)),
            pl.BlockSpec((_D, 3 * _D), lambda i: (0, 0)),
            pl.BlockSpec((1, 3 * _D), lambda i: (0, 0)),
            pl.BlockSpec((1, 3 * _D), lambda i: (0, 0)),
        ],
        out_specs=pl.BlockSpec((_BN, _D), lambda i: (i, 0)),
        out_shape=jax.ShapeDtypeStruct((_N, _D), jnp.float32),
        interpret=interpret,
    )(parts, h, wiT, whT, bi2, bh2)


@functools.lru_cache(maxsize=1)
def _make_sc_aggregate():
    mesh = plsc.VectorSubcoreMesh(core_axis_name="c", subcore_axis_name="s")

    @functools.partial(
        pl.kernel,
        out_type=jax.ShapeDtypeStruct((2, _NP, _D), jnp.float32),
        mesh=mesh,
        scratch_types=[
            pltpu.VMEM((_CPS, _C), jnp.int32),
            pltpu.VMEM((_CPS, _C), jnp.int32),
            pltpu.VMEM((_C, _D), jnp.float32),
            pltpu.VMEM((_C, _D), jnp.float32),
            pltpu.VMEM((_C, _D), jnp.float32),
            pltpu.VMEM_SHARED((_NP, _D), jnp.float32),
            pltpu.SemaphoreType.DMA,
            pltpu.SemaphoreType.DMA,
            pltpu.SemaphoreType.DMA,
            pltpu.SemaphoreType.DMA,
            pltpu.SemaphoreType.DMA,
            pltpu.SemaphoreType.DMA,
        ],
    )
    def _sc_aggregate(y_hbm, gidx_hbm, dst_hbm, zero_hbm, out_hbm,
                      gidx_v, dst_v, buf0, buf1, buf2, acc_sh,
                      g0, g1, g2, s0, s1, s2):
        cid = lax.axis_index("c")
        sid = lax.axis_index("s")
        tid = cid * 16 + sid
        bufs = (buf0, buf1, buf2)
        gsems = (g0, g1, g2)
        ssems = (s0, s1, s2)

        pltpu.sync_copy(zero_hbm, buf0)
        for z in range(_RPT // _C):
            pltpu.sync_copy(buf0, acc_sh.at[pl.ds(sid * _RPT + z * _C, _C)])
        plsc.subcore_barrier()

        def seg_body(seg, carry):
            pltpu.sync_copy(gidx_hbm.at[tid, seg], gidx_v)
            pltpu.sync_copy(dst_hbm.at[tid, seg], dst_v)

            def gather(j):
                p = j % 3
                return pltpu.async_copy(
                    y_hbm.at[gidx_v.at[j]], bufs[p], gsems[p])

            copies = [None] * _CPS
            scats = [None] * _CPS
            for j in range(3):
                copies[j] = gather(j)
            for j in range(_CPS):
                p = j % 3
                copies[j].wait()
                scats[j] = pltpu.async_copy(
                    bufs[p], acc_sh.at[dst_v.at[j]], ssems[p], add=True)
                k = j + 2
                if 3 <= k < _CPS:
                    scats[j - 1].wait()
                    copies[k] = gather(k)
            for j in range(_CPS - 3, _CPS):
                scats[j].wait()
            return carry

        lax.fori_loop(0, _NSEG, seg_body, 0)
        plsc.subcore_barrier()

        pltpu.sync_copy(acc_sh.at[pl.ds(sid * _RPT, _RPT)],
                        out_hbm.at[cid, pl.ds(sid * _RPT, _RPT)])

    return _sc_aggregate


def kernel(node_feat, edge_index, edge_feat, W_edge, b_edge,
           W_ih, W_hh, b_ih, b_hh):
    src = edge_index[0].astype(jnp.int32)
    dst = edge_index[1].astype(jnp.int32)

    ef_t = jnp.transpose(edge_feat).reshape(_NT, _E // _D, _D)
    src2 = src.reshape(_E // _D, _D)
    gidx4 = _gidx_call(ef_t, src2).reshape(_NTILES, _NSEG, _CPS, _C)
    dst4 = dst.reshape(_NTILES, _NSEG, _CPS, _C)

    wt = jnp.transpose(W_edge, (0, 2, 1))
    b3 = b_edge.reshape(_NT, 1, _D)
    wiT = jnp.transpose(W_ih)
    whT = jnp.transpose(W_hh)
    bi2 = b_ih.reshape(1, 3 * _D)
    bh2 = b_hh.reshape(1, 3 * _D)
    zeros = jnp.zeros((_C, _D), jnp.float32)

    h = node_feat
    for _ in range(_STEPS):
        y = _y_call(h, wt, b3).reshape(_NT * _N, _D)
        parts = _make_sc_aggregate()(y, gidx4, dst4, zeros)
        h = _gru_call(parts, h, wiT, whT, bi2, bh2)
    return h
